# baseline (speedup 1.0000x reference)
import functools

import jax
import jax.numpy as jnp
from jax import lax
from jax.experimental import pallas as pl
from jax.experimental.pallas import tpu as pltpu
from jax.experimental.pallas import tpu_sc as plsc

N_SPARSE = 26
N_DENSE = 13
N_OUT = N_SPARSE + N_DENSE
VOCAB = 100000
DIM = 32
B = 16384

NC = 2
NS = 16
NW = NC * NS

TOTAL_ROWS = N_SPARSE * B
PER_W = TOTAL_ROWS // NW
CHUNK = 128
NCHUNK = PER_W // CHUNK

_sc_mesh = plsc.VectorSubcoreMesh(core_axis_name="c", subcore_axis_name="s")


@functools.partial(
    pl.kernel,
    mesh=_sc_mesh,
    compiler_params=pltpu.CompilerParams(use_tc_tiling_on_sc=False),
    out_type=jax.ShapeDtypeStruct((N_OUT * B, DIM), jnp.float32),
    scratch_types=[
        pltpu.VMEM((PER_W,), jnp.int32),
        pltpu.VMEM((CHUNK, DIM), jnp.float32),
        pltpu.SemaphoreType.DMA,
    ],
)
def _sc_gather(table_hbm, idx_hbm, out_hbm, idx_v, rows_v, sem):
    wid = lax.axis_index("s") * NC + lax.axis_index("c")
    base = wid * PER_W
    pltpu.sync_copy(idx_hbm.at[pl.ds(base, PER_W)], idx_v)

    def body(ci, carry):
        off = base + ci * CHUNK
        pltpu.async_copy(
            table_hbm.at[idx_v.at[pl.ds(ci * CHUNK, CHUNK)]], rows_v, sem
        ).wait()
        pltpu.sync_copy(rows_v, out_hbm.at[pl.ds(off, CHUNK)])
        return carry

    lax.fori_loop(0, NCHUNK, body, 0)


def _dense_body(alias_ref, x_ref, w_ref, o_ref):
    del alias_ref
    o_ref[...] = x_ref[...][:, :, None] * w_ref[...]


def kernel(sparse_inputs, dense_inputs, sparse_weights, dense_weights):
    idx = sparse_inputs[:, :, 0].astype(jnp.int32)
    idx = idx + (jnp.arange(N_SPARSE, dtype=jnp.int32) * VOCAB)[:, None]
    idx_flat = idx.reshape(TOTAL_ROWS)
    table_flat = sparse_weights.reshape(N_SPARSE * VOCAB, DIM)

    out_flat = _sc_gather(table_flat, idx_flat)
    out3 = out_flat.reshape(N_OUT, B, DIM)

    x2d = dense_inputs[:, :, 0]
    bs = 1024
    nb = B // bs
    out = pl.pallas_call(
        _dense_body,
        grid=(nb,),
        in_specs=[
            pl.BlockSpec(memory_space=pl.ANY),
            pl.BlockSpec((N_DENSE, bs), lambda t: (0, t)),
            pl.BlockSpec((N_DENSE, 1, DIM), lambda t: (0, 0, 0)),
        ],
        out_specs=pl.BlockSpec((N_DENSE, bs, DIM), lambda t: (2, t, 0)),
        out_shape=jax.ShapeDtypeStruct((N_OUT, B, DIM), jnp.float32),
        input_output_aliases={0: 0},
    )(out3, x2d, dense_weights)
    return out


# speedup vs baseline: 1.0899x; 1.0899x over previous
import functools

import jax
import jax.numpy as jnp
from jax import lax
from jax.experimental import pallas as pl
from jax.experimental.pallas import tpu as pltpu
from jax.experimental.pallas import tpu_sc as plsc

N_SPARSE = 26
N_DENSE = 13
N_OUT = N_SPARSE + N_DENSE
VOCAB = 100000
DIM = 32
B = 16384
LANES = 16

NC = 2
NS = 16
NW = NC * NS

TOTAL_ROWS = N_SPARSE * B
PER_W = TOTAL_ROWS // NW
CHUNK = 128
NCHUNK = PER_W // CHUNK
NBUF = 8

DB = B // NW

_sc_mesh = plsc.VectorSubcoreMesh(core_axis_name="c", subcore_axis_name="s")


@functools.partial(
    pl.kernel,
    mesh=_sc_mesh,
    compiler_params=pltpu.CompilerParams(
        use_tc_tiling_on_sc=False, needs_layout_passes=False),
    out_type=jax.ShapeDtypeStruct((N_OUT * B, DIM), jnp.float32),
    scratch_types=[
        pltpu.VMEM((PER_W,), jnp.int32),
        pltpu.VMEM((NBUF, CHUNK, DIM), jnp.float32),
        pltpu.VMEM((N_DENSE * DB,), jnp.float32),
        pltpu.VMEM((N_DENSE * DIM,), jnp.float32),
        pltpu.VMEM((2, DB, DIM), jnp.float32),
        [pltpu.SemaphoreType.DMA] * NBUF,
        [pltpu.SemaphoreType.DMA] * NBUF,
        [pltpu.SemaphoreType.DMA] * 2,
    ],
)
def _sc_kernel(table_hbm, idx_hbm, x_hbm, w_hbm, out_hbm,
               idx_v, rows_v, x_v, w_v, dense_v, gsems, wsems, dsems):
    wid = lax.axis_index("s") * NC + lax.axis_index("c")
    base = wid * PER_W
    dbase = wid * DB

    pltpu.sync_copy(idx_hbm.at[pl.ds(base, PER_W)], idx_v)
    for j in range(N_DENSE):
        pltpu.sync_copy(x_hbm.at[pl.ds(j * B + dbase, DB)],
                        x_v.at[pl.ds(j * DB, DB)])
    pltpu.sync_copy(w_hbm, w_v)

    def start_gather(ci):
        b = ci % NBUF
        return pltpu.async_copy(
            table_hbm.at[idx_v.at[pl.ds(ci * CHUNK, CHUNK)]],
            rows_v.at[b], gsems[b])

    def start_writeback(ci):
        b = ci % NBUF
        return pltpu.async_copy(
            rows_v.at[b], out_hbm.at[pl.ds(base + ci * CHUNK, CHUNK)],
            wsems[b])

    ghandles = {}
    whandles = {}
    for ci in range(NBUF):
        ghandles[ci] = start_gather(ci)

    dhandles = {}
    for j in range(N_DENSE):
        db = j % 2
        if j >= 2:
            dhandles[j - 2].wait()
        w0 = w_v[pl.ds(j * DIM, LANES)]
        w1 = w_v[pl.ds(j * DIM + LANES, LANES)]

        def dense_body(t, carry, j=j, db=db, w0=w0, w1=w1):
            for l in range(LANES):
                pos = t * LANES + l
                src = jnp.broadcast_to(j * DB + pos, (LANES,)).astype(jnp.int32)
                xb = plsc.load_gather(x_v, [src])
                dense_v[db, pos, pl.ds(0, LANES)] = xb * w0
                dense_v[db, pos, pl.ds(LANES, LANES)] = xb * w1
            return carry

        lax.fori_loop(0, DB // LANES, dense_body, 0)
        dhandles[j] = pltpu.async_copy(
            dense_v.at[db],
            out_hbm.at[pl.ds((N_SPARSE + j) * B + dbase, DB)],
            dsems[db])

    LAG = NBUF // 2
    for ci in range(NCHUNK + LAG):
        if NBUF <= ci < NCHUNK:
            whandles[ci - NBUF].wait()
            ghandles[ci] = start_gather(ci)
        kg = ci - LAG
        if 0 <= kg < NCHUNK:
            ghandles[kg].wait()
            whandles[kg] = start_writeback(kg)
    for ci in range(NCHUNK - NBUF, NCHUNK):
        whandles[ci].wait()
    for j in (N_DENSE - 2, N_DENSE - 1):
        dhandles[j].wait()


def kernel(sparse_inputs, dense_inputs, sparse_weights, dense_weights):
    idx = sparse_inputs[:, :, 0].astype(jnp.int32)
    idx = idx + (jnp.arange(N_SPARSE, dtype=jnp.int32) * VOCAB)[:, None]
    idx_flat = idx.reshape(TOTAL_ROWS)
    table_flat = sparse_weights.reshape(N_SPARSE * VOCAB, DIM)
    x_flat = dense_inputs[:, :, 0].reshape(N_DENSE * B)
    w_flat = dense_weights.reshape(N_DENSE * DIM)

    out_flat = _sc_kernel(table_flat, idx_flat, x_flat, w_flat)
    return out_flat.reshape(N_OUT, B, DIM)
